# Initial kernel scaffold; baseline (speedup 1.0000x reference)
import jax
import jax.numpy as jnp
from jax.experimental import pallas as pl


def kernel(token_ids, embed_weight):
    raise NotImplementedError("write your pallas kernel here")


# speedup vs baseline: 6.3376x; 6.3376x over previous
import functools

import jax
import jax.numpy as jnp
from jax import lax
from jax.experimental import pallas as pl
from jax.experimental.pallas import tpu as pltpu
from jax.experimental.pallas import tpu_sc as plsc

NUM_CORES = 2
NUM_SUBCORES = 16
NUM_WORKERS = NUM_CORES * NUM_SUBCORES

CHUNK = 128


def _make_gather(total, dim):
    assert total % (NUM_WORKERS * CHUNK) == 0
    per_w = total // NUM_WORKERS
    n_chunks = per_w // CHUNK
    mesh = plsc.VectorSubcoreMesh(core_axis_name="c", subcore_axis_name="s")

    @functools.partial(
        pl.kernel,
        out_type=jax.ShapeDtypeStruct((total, dim), jnp.float32),
        mesh=mesh,
        scratch_types=[
            pltpu.VMEM((n_chunks, CHUNK), jnp.int32),
            pltpu.VMEM((CHUNK, dim), jnp.float32),
            pltpu.SemaphoreType.DMA,
        ],
    )
    def gather_kernel(idx_hbm, table_hbm, out_hbm, idx_v, rows_v, sem):
        wid = lax.axis_index("s") * NUM_CORES + lax.axis_index("c")
        pltpu.sync_copy(idx_hbm.at[pl.ds(wid * n_chunks, n_chunks)], idx_v)
        base = wid * per_w

        def body(j, carry):
            pltpu.async_copy(table_hbm.at[idx_v.at[j]], rows_v, sem).wait()
            pltpu.sync_copy(rows_v, out_hbm.at[pl.ds(base + j * CHUNK, CHUNK)])
            return carry

        lax.fori_loop(0, n_chunks, body, 0)

    return gather_kernel


def kernel(token_ids, embed_weight):
    shape = token_ids.shape
    flat = token_ids.reshape(-1).astype(jnp.int32)
    total, dim = flat.shape[0], embed_weight.shape[1]
    idx2d = flat.reshape(total // CHUNK, CHUNK)
    out = _make_gather(total, dim)(idx2d, embed_weight)
    return out.reshape(*shape, dim)


# speedup vs baseline: 9.0715x; 1.4314x over previous
import functools

import jax
import jax.numpy as jnp
from jax import lax
from jax.experimental import pallas as pl
from jax.experimental.pallas import tpu as pltpu
from jax.experimental.pallas import tpu_sc as plsc

NUM_CORES = 2
NUM_SUBCORES = 16
NUM_WORKERS = NUM_CORES * NUM_SUBCORES

CHUNK = 128
NBUF = 4


def _make_gather(total, dim):
    assert total % (NUM_WORKERS * CHUNK * NBUF) == 0
    per_w = total // NUM_WORKERS
    n_chunks = per_w // CHUNK
    n_rounds = n_chunks // NBUF
    mesh = plsc.VectorSubcoreMesh(core_axis_name="c", subcore_axis_name="s")

    @functools.partial(
        pl.kernel,
        out_type=jax.ShapeDtypeStruct((total, dim), jnp.float32),
        mesh=mesh,
        scratch_types=[
            pltpu.VMEM((n_chunks, CHUNK), jnp.int32),
            pltpu.VMEM((NBUF, CHUNK, dim), jnp.float32),
            [pltpu.SemaphoreType.DMA] * NBUF,
            [pltpu.SemaphoreType.DMA] * NBUF,
        ],
    )
    def gather_kernel(idx_hbm, table_hbm, out_hbm, idx_v, rows_v, gsem, ssem):
        wid = lax.axis_index("s") * NUM_CORES + lax.axis_index("c")
        pltpu.sync_copy(idx_hbm.at[pl.ds(wid * n_chunks, n_chunks)], idx_v)
        base = wid * per_w

        def gather(j, b):
            return pltpu.make_async_copy(
                table_hbm.at[idx_v.at[j]], rows_v.at[b], gsem[b]
            )

        def store(j, b):
            return pltpu.make_async_copy(
                rows_v.at[b], out_hbm.at[pl.ds(base + j * CHUNK, CHUNK)],
                ssem[b],
            )

        for b in range(NBUF):
            gather(b, b).start()

        def body(r, carry):
            j0 = r * NBUF
            for b in range(NBUF):
                gather(j0 + b, b).wait()
                store(j0 + b, b).start()
            for b in range(NBUF):
                store(j0 + b, b).wait()
                gather(j0 + NBUF + b, b).start()
            return carry

        lax.fori_loop(0, n_rounds - 1, body, 0)

        j0 = (n_rounds - 1) * NBUF
        for b in range(NBUF):
            gather(j0 + b, b).wait()
            store(j0 + b, b).start()
        for b in range(NBUF):
            store(j0 + b, b).wait()

    return gather_kernel


def kernel(token_ids, embed_weight):
    shape = token_ids.shape
    flat = token_ids.reshape(-1).astype(jnp.int32)
    total, dim = flat.shape[0], embed_weight.shape[1]
    idx2d = flat.reshape(total // CHUNK, CHUNK)
    out = _make_gather(total, dim)(idx2d, embed_weight)
    return out.reshape(*shape, dim)


# speedup vs baseline: 9.0941x; 1.0025x over previous
import functools

import jax
import jax.numpy as jnp
from jax import lax
from jax.experimental import pallas as pl
from jax.experimental.pallas import tpu as pltpu
from jax.experimental.pallas import tpu_sc as plsc

NUM_CORES = 2
NUM_SUBCORES = 16
NUM_WORKERS = NUM_CORES * NUM_SUBCORES

CHUNK = 128
NBUF = 5


def _make_gather(total, dim):
    assert total % (NUM_WORKERS * CHUNK * NBUF) == 0
    per_w = total // NUM_WORKERS
    n_chunks = per_w // CHUNK
    n_rounds = n_chunks // NBUF
    mesh = plsc.VectorSubcoreMesh(core_axis_name="c", subcore_axis_name="s")

    @functools.partial(
        pl.kernel,
        out_type=jax.ShapeDtypeStruct((total, dim), jnp.float32),
        mesh=mesh,
        scratch_types=[
            pltpu.VMEM((n_chunks, CHUNK), jnp.int32),
            pltpu.VMEM((NBUF, CHUNK, dim), jnp.float32),
            [pltpu.SemaphoreType.DMA] * NBUF,
            [pltpu.SemaphoreType.DMA] * NBUF,
        ],
    )
    def gather_kernel(idx_hbm, table_hbm, out_hbm, idx_v, rows_v, gsem, ssem):
        wid = lax.axis_index("s") * NUM_CORES + lax.axis_index("c")
        pltpu.sync_copy(idx_hbm.at[pl.ds(wid * n_chunks, n_chunks)], idx_v)
        base = wid * per_w

        def gather(j, b):
            return pltpu.make_async_copy(
                table_hbm.at[idx_v.at[j]], rows_v.at[b], gsem[b]
            )

        def store(j, b):
            return pltpu.make_async_copy(
                rows_v.at[b], out_hbm.at[pl.ds(base + j * CHUNK, CHUNK)],
                ssem[b],
            )

        for b in range(NBUF):
            gather(b, b).start()

        def body(r, carry):
            j0 = r * NBUF
            for b in range(NBUF):
                gather(j0 + b, b).wait()
                store(j0 + b, b).start()
            for b in range(NBUF):
                store(j0 + b, b).wait()
                gather(j0 + NBUF + b, b).start()
            return carry

        lax.fori_loop(0, n_rounds - 1, body, 0)

        j0 = (n_rounds - 1) * NBUF
        for b in range(NBUF):
            gather(j0 + b, b).wait()
            store(j0 + b, b).start()
        for b in range(NBUF):
            store(j0 + b, b).wait()

    return gather_kernel


def kernel(token_ids, embed_weight):
    shape = token_ids.shape
    flat = token_ids.reshape(-1).astype(jnp.int32)
    total, dim = flat.shape[0], embed_weight.shape[1]
    idx2d = flat.reshape(total // CHUNK, CHUNK)
    out = _make_gather(total, dim)(idx2d, embed_weight)
    return out.reshape(*shape, dim)
